# baseline (speedup 1.0000x reference)
import jax
import jax.numpy as jnp
from jax import lax
from jax.experimental import pallas as pl
from jax.experimental.pallas import tpu as pltpu
from jax.experimental.pallas import tpu_sc as plsc

B = 4
S = 8192
D = 1024
NC, NS, L = 2, 16, 16
NW = NC * NS
ROWS_PER_W = S // NW
CH = 8
NCHUNK = ROWS_PER_W // CH
G = NCHUNK * B
DL = D // L
NVEC = CH * DL


def _sc_body(x_hbm, pos_hbm, out_hbm,
             pos0, pos1, xb0, xb1, xb2, xb3,
             sin0, sin1, sin2, sin3,
             sout0, sout1, sout2, sout3,
             spos0, spos1):
    wid = lax.axis_index("s") * NC + lax.axis_index("c")
    base = wid * ROWS_PER_W

    pos_bufs = (pos0, pos1)
    x_bufs = (xb0, xb1, xb2, xb3)
    sin = (sin0, sin1, sin2, sin3)
    sout = (sout0, sout1, sout2, sout3)
    spos = (spos0, spos1)

    def start_pos(c, par):
        pltpu.make_async_copy(
            pos_hbm.at[pl.ds(base + c * CH, CH), :], pos_bufs[par], spos[par]
        ).start()

    def wait_pos(par):
        pltpu.make_async_copy(
            pos_hbm.at[pl.ds(0, CH), :], pos_bufs[par], spos[par]
        ).wait()

    def start_in(c, b, par):
        pltpu.make_async_copy(
            x_hbm.at[b, pl.ds(base + c * CH, CH), :], x_bufs[par], sin[par]
        ).start()

    def wait_in(par):
        pltpu.make_async_copy(
            x_hbm.at[0, pl.ds(0, CH), :], x_bufs[par], sin[par]
        ).wait()

    def start_out(c, b, par):
        pltpu.make_async_copy(
            x_bufs[par], out_hbm.at[b, pl.ds(base + c * CH, CH), :], sout[par]
        ).start()

    def wait_out(par):
        pltpu.make_async_copy(
            x_bufs[par], out_hbm.at[0, pl.ds(0, CH), :], sout[par]
        ).wait()

    def compute(xpar, ppar):
        xb = x_bufs[xpar]
        pb = pos_bufs[ppar]

        @plsc.parallel_loop(0, NVEC, 1, unroll=8)
        def add_body(k):
            i = k // DL
            j = (k % DL) * L
            plsc.addupdate(xb.at[i, pl.ds(j, L)], pb[i, pl.ds(j, L)])

    start_pos(0, 0)
    start_in(0, 0, 0)
    start_in(0, 1, 1)
    start_in(0, 2, 2)

    def window(m, _):
        for j in range(8):
            g = 8 * m + j
            c = 2 * m + j // 4
            b = j % 4
            xpar = j % 4
            ppar = (j // 4) % 2
            if b == 0:
                @pl.when(c + 1 < NCHUNK)
                def _():
                    start_pos(c + 1, 1 - ppar)
                wait_pos(ppar)
            j3 = j + 3
            b3 = j3 % 4
            c3 = 2 * m + j3 // 4
            npar = (xpar + 3) % 4

            @pl.when(g + 3 < G)
            def _():
                @pl.when(g >= 1)
                def _():
                    wait_out(npar)
                start_in(c3, b3, npar)
            wait_in(xpar)
            compute(xpar, ppar)
            start_out(c, b, xpar)
        return 0

    lax.fori_loop(0, NCHUNK // 2, window, 0)
    wait_out(0)
    wait_out(1)
    wait_out(2)
    wait_out(3)


def kernel(x, pos_table):
    mesh = plsc.VectorSubcoreMesh(core_axis_name="c", subcore_axis_name="s")
    k = pl.kernel(
        _sc_body,
        out_type=jax.ShapeDtypeStruct((B, S, D), jnp.float32),
        mesh=mesh,
        compiler_params=pltpu.CompilerParams(use_tc_tiling_on_sc=True),
        scratch_types=[
            pltpu.VMEM((CH, D), jnp.float32),
            pltpu.VMEM((CH, D), jnp.float32),
            pltpu.VMEM((CH, D), jnp.float32),
            pltpu.VMEM((CH, D), jnp.float32),
            pltpu.VMEM((CH, D), jnp.float32),
            pltpu.VMEM((CH, D), jnp.float32),
            pltpu.SemaphoreType.DMA,
            pltpu.SemaphoreType.DMA,
            pltpu.SemaphoreType.DMA,
            pltpu.SemaphoreType.DMA,
            pltpu.SemaphoreType.DMA,
            pltpu.SemaphoreType.DMA,
            pltpu.SemaphoreType.DMA,
            pltpu.SemaphoreType.DMA,
            pltpu.SemaphoreType.DMA,
            pltpu.SemaphoreType.DMA,
        ],
    )
    return k(x, pos_table[:S])


# speedup vs baseline: 1.0262x; 1.0262x over previous
import jax
import jax.numpy as jnp
from jax import lax
from jax.experimental import pallas as pl
from jax.experimental.pallas import tpu as pltpu
from jax.experimental.pallas import tpu_sc as plsc

B = 4
S = 8192
D = 1024
NC, NS, L = 2, 16, 16
NW = NC * NS
ROWS_PER_W = S // NW
CH = 16
NCHUNK = ROWS_PER_W // CH
G = NCHUNK * B
DL = D // L
NVEC = CH * DL


def _sc_body(x_hbm, pos_hbm, out_hbm,
             pos0, pos1, xb0, xb1, xb2, xb3,
             sin0, sin1, sin2, sin3,
             sout0, sout1, sout2, sout3,
             spos0, spos1):
    wid = lax.axis_index("s") * NC + lax.axis_index("c")
    base = wid * ROWS_PER_W

    pos_bufs = (pos0, pos1)
    x_bufs = (xb0, xb1, xb2, xb3)
    sin = (sin0, sin1, sin2, sin3)
    sout = (sout0, sout1, sout2, sout3)
    spos = (spos0, spos1)

    def start_pos(c, par):
        pltpu.make_async_copy(
            pos_hbm.at[pl.ds(base + c * CH, CH), :], pos_bufs[par], spos[par]
        ).start()

    def wait_pos(par):
        pltpu.make_async_copy(
            pos_hbm.at[pl.ds(0, CH), :], pos_bufs[par], spos[par]
        ).wait()

    def start_in(c, b, par):
        pltpu.make_async_copy(
            x_hbm.at[b, pl.ds(base + c * CH, CH), :], x_bufs[par], sin[par]
        ).start()

    def wait_in(par):
        pltpu.make_async_copy(
            x_hbm.at[0, pl.ds(0, CH), :], x_bufs[par], sin[par]
        ).wait()

    def start_out(c, b, par):
        pltpu.make_async_copy(
            x_bufs[par], out_hbm.at[b, pl.ds(base + c * CH, CH), :], sout[par]
        ).start()

    def wait_out(par):
        pltpu.make_async_copy(
            x_bufs[par], out_hbm.at[0, pl.ds(0, CH), :], sout[par]
        ).wait()

    def compute(xpar, ppar):
        xb = x_bufs[xpar]
        pb = pos_bufs[ppar]

        @plsc.parallel_loop(0, NVEC, 1, unroll=8)
        def add_body(k):
            i = k // DL
            j = (k % DL) * L
            plsc.addupdate(xb.at[i, pl.ds(j, L)], pb[i, pl.ds(j, L)])

    start_pos(0, 0)
    start_in(0, 0, 0)
    start_in(0, 1, 1)
    start_in(0, 2, 2)

    def window(m, _):
        for j in range(8):
            g = 8 * m + j
            c = 2 * m + j // 4
            b = j % 4
            xpar = j % 4
            ppar = (j // 4) % 2
            if b == 0:
                @pl.when(c + 1 < NCHUNK)
                def _():
                    start_pos(c + 1, 1 - ppar)
                wait_pos(ppar)
            j3 = j + 3
            b3 = j3 % 4
            c3 = 2 * m + j3 // 4
            npar = (xpar + 3) % 4

            @pl.when(g + 3 < G)
            def _():
                @pl.when(g >= 1)
                def _():
                    wait_out(npar)
                start_in(c3, b3, npar)
            wait_in(xpar)
            compute(xpar, ppar)
            start_out(c, b, xpar)
        return 0

    lax.fori_loop(0, NCHUNK // 2, window, 0)
    wait_out(0)
    wait_out(1)
    wait_out(2)
    wait_out(3)


def kernel(x, pos_table):
    mesh = plsc.VectorSubcoreMesh(core_axis_name="c", subcore_axis_name="s")
    k = pl.kernel(
        _sc_body,
        out_type=jax.ShapeDtypeStruct((B, S, D), jnp.float32),
        mesh=mesh,
        compiler_params=pltpu.CompilerParams(use_tc_tiling_on_sc=True),
        scratch_types=[
            pltpu.VMEM((CH, D), jnp.float32),
            pltpu.VMEM((CH, D), jnp.float32),
            pltpu.VMEM((CH, D), jnp.float32),
            pltpu.VMEM((CH, D), jnp.float32),
            pltpu.VMEM((CH, D), jnp.float32),
            pltpu.VMEM((CH, D), jnp.float32),
            pltpu.SemaphoreType.DMA,
            pltpu.SemaphoreType.DMA,
            pltpu.SemaphoreType.DMA,
            pltpu.SemaphoreType.DMA,
            pltpu.SemaphoreType.DMA,
            pltpu.SemaphoreType.DMA,
            pltpu.SemaphoreType.DMA,
            pltpu.SemaphoreType.DMA,
            pltpu.SemaphoreType.DMA,
            pltpu.SemaphoreType.DMA,
        ],
    )
    return k(x, pos_table[:S])
